# Initial kernel scaffold; baseline (speedup 1.0000x reference)
import jax
import jax.numpy as jnp
from jax.experimental import pallas as pl


def kernel(x, edge_index, edge_weight, only_z, W_in, b_in, W0, b0, lnw0, lnb0, W1, b1, lnw1, lnb1, W2, b2, lnw2, lnb2, out_lnw, out_lnb, W_out, b_out):
    raise NotImplementedError("write your pallas kernel here")


# speedup vs baseline: 5.8257x; 5.8257x over previous
import functools

import jax
import jax.numpy as jnp
from jax import lax
from jax.experimental import pallas as pl
from jax.experimental.pallas import tpu as pltpu
from jax.experimental.pallas import tpu_sc as plsc

N = 10000
E = 320000
F = 128
NCLASS = 64

NC = 2
NS = 16
NW = NC * NS
EPW = E // NW
C = 80
NCHUNK = EPW // C
SCH = 25
NSUP = NCHUNK // SCH
NPAD = 10240
ROWS_PT = NPAD // NS


def _i16(v):
    return jnp.full((16,), v, dtype=jnp.int32)


def _spmm_body(hn, srcr, dstr, wr, zeros, out, acc, src_v, dst_v, w_v, rows_v, sem):
    c = lax.axis_index("c")
    s = lax.axis_index("s")
    wid = s * NC + c

    pltpu.sync_copy(zeros.at[pl.ds(s * ROWS_PT, ROWS_PT)],
                    acc.at[pl.ds(s * ROWS_PT, ROWS_PT)])
    plsc.subcore_barrier()

    def superchunk(sc_i, carry0):
        pltpu.sync_copy(srcr.at[wid, sc_i], src_v)
        pltpu.sync_copy(dstr.at[wid, sc_i], dst_v)
        pltpu.sync_copy(wr.at[wid, sc_i], w_v)

        def chunk(k, carry):
            pltpu.async_copy(hn.at[src_v.at[k]], rows_v, sem).wait()

            def group(g, carry2):
                w16 = w_v[k, pl.ds(g * 16, 16)]
                for j in range(16):
                    wvec = jnp.full((16,), w16[j], dtype=jnp.float32)
                    e = g * 16 + j
                    for f in range(F // 16):
                        sl = (e, pl.ds(f * 16, 16))
                        rows_v[sl] = rows_v[sl] * wvec
                return carry2

            lax.fori_loop(0, C // 16, group, 0, unroll=False)
            pltpu.sync_copy(rows_v, acc.at[dst_v.at[k]], add=True)
            return carry

        lax.fori_loop(0, SCH, chunk, 0, unroll=False)
        return carry0

    lax.fori_loop(0, NSUP, superchunk, 0, unroll=False)
    plsc.subcore_barrier()
    pltpu.sync_copy(acc.at[pl.ds(s * ROWS_PT, ROWS_PT)],
                    out.at[c, pl.ds(s * ROWS_PT, ROWS_PT)])


_spmm = pl.kernel(
    _spmm_body,
    out_type=jax.ShapeDtypeStruct((NC, NPAD, F), jnp.float32),
    mesh=plsc.VectorSubcoreMesh(core_axis_name="c", subcore_axis_name="s"),
    scratch_types=[
        pltpu.VMEM_SHARED((NPAD, F), jnp.float32),
        pltpu.VMEM((SCH, C), jnp.int32),
        pltpu.VMEM((SCH, C), jnp.int32),
        pltpu.VMEM((SCH, C), jnp.float32),
        pltpu.VMEM((C, F), jnp.float32),
        pltpu.SemaphoreType.DMA,
    ],
)


def _layernorm(h, w, b):
    m = jnp.mean(h, axis=-1, keepdims=True)
    v = jnp.mean((h - m) * (h - m), axis=-1, keepdims=True)
    return (h - m) * lax.rsqrt(v + 1e-5) * w + b


def _tc_in_body(x_ref, w_ref, b_ref, lnw_ref, lnb_ref, h_ref, hn_ref):
    h = jnp.dot(x_ref[...], w_ref[...], preferred_element_type=jnp.float32)
    h = jnp.maximum(h + b_ref[...], 0.0)
    h_ref[...] = h
    hn_ref[...] = _layernorm(h, lnw_ref[...], lnb_ref[...])


def _tc_mid_body(s2_ref, h_ref, w_ref, b_ref, lnw_ref, lnb_ref, ho_ref, hn_ref):
    sp = s2_ref[0] + s2_ref[1]
    t = jnp.dot(sp, w_ref[...], preferred_element_type=jnp.float32)
    t = jnp.maximum(t + b_ref[...], 0.0)
    h = h_ref[...] + t
    ho_ref[...] = h
    hn_ref[...] = _layernorm(h, lnw_ref[...], lnb_ref[...])


def _tc_last_body(s2_ref, h_ref, w_ref, b_ref, lnw_ref, lnb_ref,
                  wout_ref, bout_ref, z_ref):
    sp = s2_ref[0] + s2_ref[1]
    t = jnp.dot(sp, w_ref[...], preferred_element_type=jnp.float32) + b_ref[...]
    h = h_ref[...] + t
    hn = _layernorm(h, lnw_ref[...], lnb_ref[...])
    z_ref[...] = jnp.dot(hn, wout_ref[...],
                         preferred_element_type=jnp.float32) + bout_ref[...]


_BLK = 400
_GRID = N // _BLK


def _rowspec(width=F):
    return pl.BlockSpec((_BLK, width), lambda i: (i, 0))


def _fullspec(shape):
    return pl.BlockSpec(shape, lambda i: tuple(0 for _ in shape))


_tc_in = pl.pallas_call(
    _tc_in_body,
    grid=(_GRID,),
    in_specs=[_rowspec(), _fullspec((F, F)), _fullspec((1, F)),
              _fullspec((1, F)), _fullspec((1, F))],
    out_specs=[_rowspec(), _rowspec()],
    out_shape=[jax.ShapeDtypeStruct((N, F), jnp.float32),
               jax.ShapeDtypeStruct((N, F), jnp.float32)],
)

_tc_mid = pl.pallas_call(
    _tc_mid_body,
    grid=(_GRID,),
    in_specs=[pl.BlockSpec((NC, _BLK, F), lambda i: (0, i, 0)),
              _rowspec(), _fullspec((F, F)), _fullspec((1, F)),
              _fullspec((1, F)), _fullspec((1, F))],
    out_specs=[_rowspec(), _rowspec()],
    out_shape=[jax.ShapeDtypeStruct((N, F), jnp.float32),
               jax.ShapeDtypeStruct((N, F), jnp.float32)],
)

_tc_last = pl.pallas_call(
    _tc_last_body,
    grid=(_GRID,),
    in_specs=[pl.BlockSpec((NC, _BLK, F), lambda i: (0, i, 0)),
              _rowspec(), _fullspec((F, F)), _fullspec((1, F)),
              _fullspec((1, F)), _fullspec((1, F)),
              _fullspec((F, NCLASS)), _fullspec((1, NCLASS))],
    out_specs=pl.BlockSpec((_BLK, NCLASS), lambda i: (i, 0)),
    out_shape=jax.ShapeDtypeStruct((N, NCLASS), jnp.float32),
)


def kernel(x, edge_index, edge_weight, only_z, W_in, b_in, W0, b0, lnw0, lnb0,
           W1, b1, lnw1, lnb1, W2, b2, lnw2, lnb2, out_lnw, out_lnb,
           W_out, b_out):
    srcr = edge_index[0].reshape(NW, NSUP, SCH, C)
    dstr = edge_index[1].reshape(NW, NSUP, SCH, C)
    wr = edge_weight.reshape(NW, NSUP, SCH, C)
    zeros = jnp.zeros((NPAD, F), jnp.float32)

    r2 = lambda v: v.reshape(1, -1)
    h, hn = _tc_in(x, W_in, r2(b_in), r2(lnw0), r2(lnb0))

    s2 = _spmm(hn, srcr, dstr, wr, zeros)
    h, hn = _tc_mid(s2, h, W0, r2(b0), r2(lnw1), r2(lnb1))

    s2 = _spmm(hn, srcr, dstr, wr, zeros)
    h, hn = _tc_mid(s2, h, W1, r2(b1), r2(lnw2), r2(lnb2))

    s2 = _spmm(hn, srcr, dstr, wr, zeros)
    z = _tc_last(s2, h, W2, r2(b2), r2(out_lnw), r2(out_lnb),
                 W_out, r2(b_out))
    return z * jnp.asarray(only_z, dtype=z.dtype)
